# Initial kernel scaffold; baseline (speedup 1.0000x reference)
import jax
import jax.numpy as jnp
from jax.experimental import pallas as pl


def kernel(x, edge_index, W1, a1_src, a1_dst, b1, W2l, W2r, b2, W3l, W3r, b3):
    raise NotImplementedError("write your pallas kernel here")


# speedup vs baseline: 1.7492x; 1.7492x over previous
import functools

import jax
import jax.numpy as jnp
from jax import lax
from jax.experimental import pallas as pl
from jax.experimental.pallas import tpu as pltpu

N = 10000
E = 640000
D_IN = 128
D_H1 = 256
D_H2 = 128
D_OUT = 128

_R = 1000


def _mm1_body(agg_ref, invs_ref, W1_ref, b1_ref, W2l_ref, W2r_ref,
              p2_ref, r2_ref):
    agg = agg_ref[...] * invs_ref[...]
    h2 = jnp.maximum(jnp.dot(agg, W1_ref[...],
                             preferred_element_type=jnp.float32)
                     + b1_ref[...], 0.0)
    p2_ref[...] = jnp.dot(h2, W2l_ref[...], preferred_element_type=jnp.float32)
    r2_ref[...] = jnp.dot(h2, W2r_ref[...], preferred_element_type=jnp.float32)


def _tc_layer1(agg1, inv_s, W1, b1, W2l, W2r):
    grid = (N // _R,)
    return pl.pallas_call(
        _mm1_body,
        grid=grid,
        in_specs=[
            pl.BlockSpec((_R, D_IN), lambda i: (i, 0)),
            pl.BlockSpec((_R, 1), lambda i: (i, 0)),
            pl.BlockSpec((D_IN, D_H1), lambda i: (0, 0)),
            pl.BlockSpec((1, D_H1), lambda i: (0, 0)),
            pl.BlockSpec((D_H1, D_H2), lambda i: (0, 0)),
            pl.BlockSpec((D_H1, D_H2), lambda i: (0, 0)),
        ],
        out_specs=[
            pl.BlockSpec((_R, D_H2), lambda i: (i, 0)),
            pl.BlockSpec((_R, D_H2), lambda i: (i, 0)),
        ],
        out_shape=[
            jax.ShapeDtypeStruct((N, D_H2), jnp.float32),
            jax.ShapeDtypeStruct((N, D_H2), jnp.float32),
        ],
    )(agg1, inv_s, W1, b1, W2l, W2r)


def _mm2_body(agg_ref, invd_ref, r_ref, b_ref, Wl_ref, Wr_ref,
              p_ref, rn_ref):
    out = jnp.maximum(agg_ref[...] * invd_ref[...] + r_ref[...] + b_ref[...],
                      0.0)
    p_ref[...] = jnp.dot(out, Wl_ref[...], preferred_element_type=jnp.float32)
    rn_ref[...] = jnp.dot(out, Wr_ref[...], preferred_element_type=jnp.float32)


def _tc_layer2(agg2, invdeg, r2, b2, W3l, W3r):
    grid = (N // _R,)
    return pl.pallas_call(
        _mm2_body,
        grid=grid,
        in_specs=[
            pl.BlockSpec((_R, D_H2), lambda i: (i, 0)),
            pl.BlockSpec((_R, 1), lambda i: (i, 0)),
            pl.BlockSpec((_R, D_H2), lambda i: (i, 0)),
            pl.BlockSpec((1, D_H2), lambda i: (0, 0)),
            pl.BlockSpec((D_H2, D_OUT), lambda i: (0, 0)),
            pl.BlockSpec((D_H2, D_OUT), lambda i: (0, 0)),
        ],
        out_specs=[
            pl.BlockSpec((_R, D_OUT), lambda i: (i, 0)),
            pl.BlockSpec((_R, D_OUT), lambda i: (i, 0)),
        ],
        out_shape=[
            jax.ShapeDtypeStruct((N, D_OUT), jnp.float32),
            jax.ShapeDtypeStruct((N, D_OUT), jnp.float32),
        ],
    )(agg2, invdeg, r2, b2, W3l, W3r)


def _mm3_body(agg_ref, invd_ref, r_ref, b_ref, out_ref):
    out_ref[...] = agg_ref[...] * invd_ref[...] + r_ref[...] + b_ref[...]


def _tc_layer3(agg3, invdeg, r3, b3):
    grid = (N // _R,)
    return pl.pallas_call(
        _mm3_body,
        grid=grid,
        in_specs=[
            pl.BlockSpec((_R, D_OUT), lambda i: (i, 0)),
            pl.BlockSpec((_R, 1), lambda i: (i, 0)),
            pl.BlockSpec((_R, D_OUT), lambda i: (i, 0)),
            pl.BlockSpec((1, D_OUT), lambda i: (0, 0)),
        ],
        out_specs=pl.BlockSpec((_R, D_OUT), lambda i: (i, 0)),
        out_shape=jax.ShapeDtypeStruct((N, D_OUT), jnp.float32),
    )(agg3, invdeg, r3, b3)


def kernel(x, edge_index, W1, a1_src, a1_dst, b1, W2l, W2r, b2, W3l, W3r, b3):
    src = edge_index[0]
    dst = edge_index[1]

    va = W1 @ a1_src
    vb = W1 @ a1_dst
    al = x @ va
    ar = x @ vb

    e = al[src] + ar[dst]
    e = jnp.maximum(e, 0.2 * e)
    ex = jnp.exp(e)
    s = jax.ops.segment_sum(ex, dst, num_segments=N)
    deg = jax.ops.segment_sum(jnp.ones((E,), jnp.float32), dst, num_segments=N)
    inv_s = (1.0 / (s + 1e-16))[:, None]
    invdeg = (1.0 / jnp.maximum(deg, 1.0))[:, None]

    agg1 = jax.ops.segment_sum(ex[:, None] * x[src], dst, num_segments=N)
    p2, r2 = _tc_layer1(agg1, inv_s, W1, b1[None, :], W2l, W2r)

    agg2 = jax.ops.segment_sum(p2[src], dst, num_segments=N)
    p3, r3 = _tc_layer2(agg2, invdeg, r2, b2[None, :], W3l, W3r)

    agg3 = jax.ops.segment_sum(p3[src], dst, num_segments=N)
    return _tc_layer3(agg3, invdeg, r3, b3[None, :])


# speedup vs baseline: 2.3158x; 1.3239x over previous
import functools

import jax
import jax.numpy as jnp
from jax import lax
from jax.experimental import pallas as pl
from jax.experimental.pallas import tpu as pltpu
from jax.experimental.pallas import tpu_sc as plsc

N = 10000
E = 640000
D_IN = 128
D_H1 = 256
D_H2 = 128
D_OUT = 128

_NC = 2
_NS = 16
_NW = _NC * _NS
_CH = 128
_SUBW = 160
_SS = 8
_NSUP = _SUBW // _SS
_NSUPL = 5
_EROWS = _NW * _SUBW
_NPAD = 10112
_DROWS = _NPAD // _NS

_R = 632


def _spmm_sc_body(p_hbm, src_hbm, dst_hbm, zero_hbm, out_hbm,
                  src_v, dst_v, rows_v, acc_sh, sem):
    c = lax.axis_index("c")
    s = lax.axis_index("s")
    w = c * _NS + s
    pltpu.sync_copy(zero_hbm.at[pl.ds(s * _DROWS, _DROWS), :],
                    acc_sh.at[pl.ds(s * _DROWS, _DROWS), :])
    plsc.subcore_barrier()
    nsuper = jnp.where(w == _NW - 1, _NSUPL, _NSUP)

    def outer(g, carry):
        r0 = w * _SUBW + g * _SS
        pltpu.sync_copy(src_hbm.at[pl.ds(r0, _SS), :], src_v)
        pltpu.sync_copy(dst_hbm.at[pl.ds(r0, _SS), :], dst_v)

        def inner(j, carry2):
            pltpu.async_copy(p_hbm.at[src_v.at[j]], rows_v, sem).wait()
            pltpu.sync_copy(rows_v, acc_sh.at[dst_v.at[j]], add=True)
            return carry2

        lax.fori_loop(0, _SS, inner, 0)
        return carry

    lax.fori_loop(0, nsuper, outer, 0)
    plsc.subcore_barrier()
    pltpu.sync_copy(acc_sh.at[pl.ds(s * _DROWS, _DROWS), :],
                    out_hbm.at[c, pl.ds(s * _DROWS, _DROWS), :])


@jax.jit
def _spmm_sc(p, src2d, dst2d, zeros2d):
    f = pl.kernel(
        _spmm_sc_body,
        out_type=jax.ShapeDtypeStruct((_NC, _NPAD, D_H2), jnp.float32),
        mesh=plsc.VectorSubcoreMesh(core_axis_name="c", subcore_axis_name="s"),
        scratch_types=[
            pltpu.VMEM((_SS, _CH), jnp.int32),
            pltpu.VMEM((_SS, _CH), jnp.int32),
            pltpu.VMEM((_CH, D_H2), jnp.float32),
            pltpu.VMEM_SHARED((_NPAD, D_H2), jnp.float32),
            pltpu.SemaphoreType.DMA,
        ],
    )
    return f(p, src2d, dst2d, zeros2d)


def _mm1_body(a0_ref, a1_ref, invs_ref, W1_ref, b1_ref, W2l_ref, W2r_ref,
              p2_ref, r2_ref):
    agg = (a0_ref[0] + a1_ref[0]) * invs_ref[...]
    h2 = jnp.maximum(jnp.dot(agg, W1_ref[...],
                             preferred_element_type=jnp.float32)
                     + b1_ref[...], 0.0)
    p2_ref[...] = jnp.dot(h2, W2l_ref[...], preferred_element_type=jnp.float32)
    r2_ref[...] = jnp.dot(h2, W2r_ref[...], preferred_element_type=jnp.float32)


def _tc_layer1(agg1p, inv_s, W1, b1, W2l, W2r):
    return pl.pallas_call(
        _mm1_body,
        grid=(_NPAD // _R,),
        in_specs=[
            pl.BlockSpec((1, _R, D_IN), lambda i: (0, i, 0)),
            pl.BlockSpec((1, _R, D_IN), lambda i: (1, i, 0)),
            pl.BlockSpec((_R, 1), lambda i: (i, 0)),
            pl.BlockSpec((D_IN, D_H1), lambda i: (0, 0)),
            pl.BlockSpec((1, D_H1), lambda i: (0, 0)),
            pl.BlockSpec((D_H1, D_H2), lambda i: (0, 0)),
            pl.BlockSpec((D_H1, D_H2), lambda i: (0, 0)),
        ],
        out_specs=[
            pl.BlockSpec((_R, D_H2), lambda i: (i, 0)),
            pl.BlockSpec((_R, D_H2), lambda i: (i, 0)),
        ],
        out_shape=[
            jax.ShapeDtypeStruct((_NPAD, D_H2), jnp.float32),
            jax.ShapeDtypeStruct((_NPAD, D_H2), jnp.float32),
        ],
    )(agg1p, agg1p, inv_s, W1, b1, W2l, W2r)


def _mm2_body(a0_ref, a1_ref, invd_ref, r_ref, b_ref, Wl_ref, Wr_ref,
              p_ref, rn_ref):
    out = jnp.maximum((a0_ref[0] + a1_ref[0]) * invd_ref[...]
                      + r_ref[...] + b_ref[...], 0.0)
    p_ref[...] = jnp.dot(out, Wl_ref[...], preferred_element_type=jnp.float32)
    rn_ref[...] = jnp.dot(out, Wr_ref[...], preferred_element_type=jnp.float32)


def _tc_layer2(agg2p, invdeg, r2, b2, W3l, W3r):
    return pl.pallas_call(
        _mm2_body,
        grid=(_NPAD // _R,),
        in_specs=[
            pl.BlockSpec((1, _R, D_H2), lambda i: (0, i, 0)),
            pl.BlockSpec((1, _R, D_H2), lambda i: (1, i, 0)),
            pl.BlockSpec((_R, 1), lambda i: (i, 0)),
            pl.BlockSpec((_R, D_H2), lambda i: (i, 0)),
            pl.BlockSpec((1, D_H2), lambda i: (0, 0)),
            pl.BlockSpec((D_H2, D_OUT), lambda i: (0, 0)),
            pl.BlockSpec((D_H2, D_OUT), lambda i: (0, 0)),
        ],
        out_specs=[
            pl.BlockSpec((_R, D_OUT), lambda i: (i, 0)),
            pl.BlockSpec((_R, D_OUT), lambda i: (i, 0)),
        ],
        out_shape=[
            jax.ShapeDtypeStruct((_NPAD, D_OUT), jnp.float32),
            jax.ShapeDtypeStruct((_NPAD, D_OUT), jnp.float32),
        ],
    )(agg2p, agg2p, invdeg, r2, b2, W3l, W3r)


def _mm3_body(a0_ref, a1_ref, invd_ref, r_ref, b_ref, out_ref):
    out_ref[...] = ((a0_ref[0] + a1_ref[0]) * invd_ref[...]
                    + r_ref[...] + b_ref[...])


def _tc_layer3(agg3p, invdeg, r3, b3):
    return pl.pallas_call(
        _mm3_body,
        grid=(_NPAD // _R,),
        in_specs=[
            pl.BlockSpec((1, _R, D_OUT), lambda i: (0, i, 0)),
            pl.BlockSpec((1, _R, D_OUT), lambda i: (1, i, 0)),
            pl.BlockSpec((_R, 1), lambda i: (i, 0)),
            pl.BlockSpec((_R, D_OUT), lambda i: (i, 0)),
            pl.BlockSpec((1, D_OUT), lambda i: (0, 0)),
        ],
        out_specs=pl.BlockSpec((_R, D_OUT), lambda i: (i, 0)),
        out_shape=jax.ShapeDtypeStruct((_NPAD, D_OUT), jnp.float32),
    )(agg3p, agg3p, invdeg, r3, b3)


def kernel(x, edge_index, W1, a1_src, a1_dst, b1, W2l, W2r, b2, W3l, W3r, b3):
    src = edge_index[0]
    dst = edge_index[1]
    pad = _EROWS * _CH - E
    src2d = jnp.pad(src, (0, pad)).reshape(_EROWS, _CH)
    dst2d = jnp.pad(dst, (0, pad)).reshape(_EROWS, _CH)
    zeros2d = jnp.zeros((_NPAD, D_H2), jnp.float32)

    va = W1 @ a1_src
    vb = W1 @ a1_dst
    al = x @ va
    ar = x @ vb

    e = al[src] + ar[dst]
    e = jnp.maximum(e, 0.2 * e)
    ex = jnp.exp(e)
    s = jax.ops.segment_sum(ex, dst, num_segments=N)
    deg = jax.ops.segment_sum(jnp.ones((E,), jnp.float32), dst, num_segments=N)
    inv_s = jnp.pad((1.0 / (s + 1e-16)), (0, _NPAD - N))[:, None]
    invdeg = jnp.pad((1.0 / jnp.maximum(deg, 1.0)), (0, _NPAD - N))[:, None]

    agg1 = jax.ops.segment_sum(ex[:, None] * x[src], dst, num_segments=N)
    agg1p = jnp.zeros((_NC, _NPAD, D_IN), jnp.float32)
    agg1p = agg1p.at[0, :N].set(agg1)

    p2, r2 = _tc_layer1(agg1p, inv_s, W1, b1[None, :], W2l, W2r)

    agg2p = _spmm_sc(p2, src2d, dst2d, zeros2d)
    p3, r3 = _tc_layer2(agg2p, invdeg, r2, b2[None, :], W3l, W3r)

    agg3p = _spmm_sc(p3, src2d, dst2d, zeros2d)
    out3 = _tc_layer3(agg3p, invdeg, r3, b3[None, :])
    return out3[:N]


# speedup vs baseline: 31.8499x; 13.7536x over previous
import functools

import jax
import jax.numpy as jnp
from jax import lax
from jax.experimental import pallas as pl
from jax.experimental.pallas import tpu as pltpu
from jax.experimental.pallas import tpu_sc as plsc

N = 10000
E = 640000
D_IN = 128
D_H1 = 256
D_H2 = 128
D_OUT = 128

_NC = 2
_NS = 16
_NW = _NC * _NS
_CH = 128
_SUBW = 160
_SS = 8
_NSUP = _SUBW // _SS
_NSUPL = 5
_EROWS = _NW * _SUBW
_NPAD = 10112
_DROWS = _NPAD // _NS

_R = 632
_SPAD = 10240
_SD = 640


def _spmm_sc_body(p_hbm, src_hbm, dst_hbm, zero_hbm, out_hbm,
                  src_v, dst_v, rows_v, acc_sh, sem):
    c = lax.axis_index("c")
    s = lax.axis_index("s")
    w = c * _NS + s
    pltpu.sync_copy(zero_hbm.at[pl.ds(s * _DROWS, _DROWS), :],
                    acc_sh.at[pl.ds(s * _DROWS, _DROWS), :])
    plsc.subcore_barrier()
    nsuper = jnp.where(w == _NW - 1, _NSUPL, _NSUP)

    def outer(g, carry):
        r0 = w * _SUBW + g * _SS
        pltpu.sync_copy(src_hbm.at[pl.ds(r0, _SS), :], src_v)
        pltpu.sync_copy(dst_hbm.at[pl.ds(r0, _SS), :], dst_v)

        def inner(j, carry2):
            pltpu.async_copy(p_hbm.at[src_v.at[j]], rows_v, sem).wait()
            pltpu.sync_copy(rows_v, acc_sh.at[dst_v.at[j]], add=True)
            return carry2

        lax.fori_loop(0, _SS, inner, 0)
        return carry

    lax.fori_loop(0, nsuper, outer, 0)
    plsc.subcore_barrier()
    pltpu.sync_copy(acc_sh.at[pl.ds(s * _DROWS, _DROWS), :],
                    out_hbm.at[c, pl.ds(s * _DROWS, _DROWS), :])


def _gat_sc_body(x_hbm, al_hbm, ar_hbm, src_hbm, dst_hbm, z2_hbm, z1_hbm,
                 agg_hbm, s0_hbm, s1_hbm, d0_hbm, d1_hbm,
                 al_v, ar_v, src_s, dst_s, rows_v, exb, ones_v,
                 acc_sh, s_sh, deg_sh, sem):
    c = lax.axis_index("c")
    s = lax.axis_index("s")
    w = c * _NS + s
    pltpu.sync_copy(z2_hbm.at[pl.ds(s * _DROWS, _DROWS), :],
                    acc_sh.at[pl.ds(s * _DROWS, _DROWS), :])
    pltpu.sync_copy(z1_hbm.at[pl.ds(s * _SD, _SD)],
                    s_sh.at[pl.ds(s * _SD, _SD)])
    pltpu.sync_copy(z1_hbm.at[pl.ds(s * _SD, _SD)],
                    deg_sh.at[pl.ds(s * _SD, _SD)])
    pltpu.sync_copy(al_hbm, al_v)
    pltpu.sync_copy(ar_hbm, ar_v)
    for k in range(_CH // 16):
        ones_v[pl.ds(k * 16, 16)] = jnp.ones((16,), jnp.float32)
    plsc.subcore_barrier()
    nsuper = jnp.where(w == _NW - 1, _NSUPL, _NSUP)

    def outer(g, carry):
        r0 = w * _SUBW + g * _SS
        pltpu.sync_copy(src_hbm.at[pl.ds(r0, _SS), :], src_s)
        pltpu.sync_copy(dst_hbm.at[pl.ds(r0, _SS), :], dst_s)

        def inner(j, carry2):
            pltpu.async_copy(x_hbm.at[src_s.at[j]], rows_v, sem).wait()
            for k in range(_CH // 16):
                sl = pl.ds(k * 16, 16)
                sv = src_s[j, sl]
                dv = dst_s[j, sl]
                e = plsc.load_gather(al_v, [sv]) + plsc.load_gather(ar_v, [dv])
                e = jnp.maximum(e, 0.2 * e)
                exv = jnp.exp(e)
                exb[sl] = exv
                for l in range(16):
                    wl = exv[l]
                    r = k * 16 + l
                    for m in range(D_IN // 16):
                        slm = pl.ds(m * 16, 16)
                        rows_v[r, slm] = rows_v[r, slm] * wl
            pltpu.sync_copy(exb, s_sh.at[dst_s.at[j]], add=True)
            pltpu.sync_copy(ones_v, deg_sh.at[dst_s.at[j]], add=True)
            pltpu.sync_copy(rows_v, acc_sh.at[dst_s.at[j]], add=True)
            return carry2

        lax.fori_loop(0, _SS, inner, 0)
        return carry

    lax.fori_loop(0, nsuper, outer, 0)
    plsc.subcore_barrier()
    pltpu.sync_copy(acc_sh.at[pl.ds(s * _DROWS, _DROWS), :],
                    agg_hbm.at[c, pl.ds(s * _DROWS, _DROWS), :])
    sl1 = pl.ds(s * _SD, _SD)

    @pl.when(c == 0)
    def _dump0():
        pltpu.sync_copy(s_sh.at[sl1], s0_hbm.at[sl1])
        pltpu.sync_copy(deg_sh.at[sl1], d0_hbm.at[sl1])

    @pl.when(c == 1)
    def _dump1():
        pltpu.sync_copy(s_sh.at[sl1], s1_hbm.at[sl1])
        pltpu.sync_copy(deg_sh.at[sl1], d1_hbm.at[sl1])


@jax.jit
def _gat_sc(x, al, ar, src2d, dst2d, zeros2d, zeros1d):
    f = pl.kernel(
        _gat_sc_body,
        out_type=[
            jax.ShapeDtypeStruct((_NC, _NPAD, D_IN), jnp.float32),
            jax.ShapeDtypeStruct((_SPAD,), jnp.float32),
            jax.ShapeDtypeStruct((_SPAD,), jnp.float32),
            jax.ShapeDtypeStruct((_SPAD,), jnp.float32),
            jax.ShapeDtypeStruct((_SPAD,), jnp.float32),
        ],
        mesh=plsc.VectorSubcoreMesh(core_axis_name="c", subcore_axis_name="s"),
        compiler_params=pltpu.CompilerParams(needs_layout_passes=False),
        scratch_types=[
            pltpu.VMEM((N,), jnp.float32),
            pltpu.VMEM((N,), jnp.float32),
            pltpu.VMEM((_SS, _CH), jnp.int32),
            pltpu.VMEM((_SS, _CH), jnp.int32),
            pltpu.VMEM((_CH, D_IN), jnp.float32),
            pltpu.VMEM((_CH,), jnp.float32),
            pltpu.VMEM((_CH,), jnp.float32),
            pltpu.VMEM_SHARED((_NPAD, D_IN), jnp.float32),
            pltpu.VMEM_SHARED((_SPAD,), jnp.float32),
            pltpu.VMEM_SHARED((_SPAD,), jnp.float32),
            pltpu.SemaphoreType.DMA,
        ],
    )
    return f(x, al, ar, src2d, dst2d, zeros2d, zeros1d)


@jax.jit
def _spmm_sc(p, src2d, dst2d, zeros2d):
    f = pl.kernel(
        _spmm_sc_body,
        out_type=jax.ShapeDtypeStruct((_NC, _NPAD, D_H2), jnp.float32),
        mesh=plsc.VectorSubcoreMesh(core_axis_name="c", subcore_axis_name="s"),
        scratch_types=[
            pltpu.VMEM((_SS, _CH), jnp.int32),
            pltpu.VMEM((_SS, _CH), jnp.int32),
            pltpu.VMEM((_CH, D_H2), jnp.float32),
            pltpu.VMEM_SHARED((_NPAD, D_H2), jnp.float32),
            pltpu.SemaphoreType.DMA,
        ],
    )
    return f(p, src2d, dst2d, zeros2d)


def _mm0_body(x_ref, V_ref, out_ref):
    out_ref[...] = jnp.dot(x_ref[...], V_ref[...],
                           preferred_element_type=jnp.float32)


def _tc_proj(x, Vp):
    return pl.pallas_call(
        _mm0_body,
        grid=(N // 1000,),
        in_specs=[
            pl.BlockSpec((1000, D_IN), lambda i: (i, 0)),
            pl.BlockSpec((D_IN, D_IN), lambda i: (0, 0)),
        ],
        out_specs=pl.BlockSpec((1000, D_IN), lambda i: (i, 0)),
        out_shape=jax.ShapeDtypeStruct((N, D_IN), jnp.float32),
    )(x, Vp)


def _mm1_body(a0_ref, a1_ref, invs_ref, W1_ref, b1_ref, W2l_ref, W2r_ref,
              p2_ref, r2_ref):
    agg = (a0_ref[0] + a1_ref[0]) * invs_ref[...]
    h2 = jnp.maximum(jnp.dot(agg, W1_ref[...],
                             preferred_element_type=jnp.float32)
                     + b1_ref[...], 0.0)
    p2_ref[...] = jnp.dot(h2, W2l_ref[...], preferred_element_type=jnp.float32)
    r2_ref[...] = jnp.dot(h2, W2r_ref[...], preferred_element_type=jnp.float32)


def _tc_layer1(agg1p, inv_s, W1, b1, W2l, W2r):
    return pl.pallas_call(
        _mm1_body,
        grid=(_NPAD // _R,),
        in_specs=[
            pl.BlockSpec((1, _R, D_IN), lambda i: (0, i, 0)),
            pl.BlockSpec((1, _R, D_IN), lambda i: (1, i, 0)),
            pl.BlockSpec((_R, 1), lambda i: (i, 0)),
            pl.BlockSpec((D_IN, D_H1), lambda i: (0, 0)),
            pl.BlockSpec((1, D_H1), lambda i: (0, 0)),
            pl.BlockSpec((D_H1, D_H2), lambda i: (0, 0)),
            pl.BlockSpec((D_H1, D_H2), lambda i: (0, 0)),
        ],
        out_specs=[
            pl.BlockSpec((_R, D_H2), lambda i: (i, 0)),
            pl.BlockSpec((_R, D_H2), lambda i: (i, 0)),
        ],
        out_shape=[
            jax.ShapeDtypeStruct((_NPAD, D_H2), jnp.float32),
            jax.ShapeDtypeStruct((_NPAD, D_H2), jnp.float32),
        ],
    )(agg1p, agg1p, inv_s, W1, b1, W2l, W2r)


def _mm2_body(a0_ref, a1_ref, invd_ref, r_ref, b_ref, Wl_ref, Wr_ref,
              p_ref, rn_ref):
    out = jnp.maximum((a0_ref[0] + a1_ref[0]) * invd_ref[...]
                      + r_ref[...] + b_ref[...], 0.0)
    p_ref[...] = jnp.dot(out, Wl_ref[...], preferred_element_type=jnp.float32)
    rn_ref[...] = jnp.dot(out, Wr_ref[...], preferred_element_type=jnp.float32)


def _tc_layer2(agg2p, invdeg, r2, b2, W3l, W3r):
    return pl.pallas_call(
        _mm2_body,
        grid=(_NPAD // _R,),
        in_specs=[
            pl.BlockSpec((1, _R, D_H2), lambda i: (0, i, 0)),
            pl.BlockSpec((1, _R, D_H2), lambda i: (1, i, 0)),
            pl.BlockSpec((_R, 1), lambda i: (i, 0)),
            pl.BlockSpec((_R, D_H2), lambda i: (i, 0)),
            pl.BlockSpec((1, D_H2), lambda i: (0, 0)),
            pl.BlockSpec((D_H2, D_OUT), lambda i: (0, 0)),
            pl.BlockSpec((D_H2, D_OUT), lambda i: (0, 0)),
        ],
        out_specs=[
            pl.BlockSpec((_R, D_OUT), lambda i: (i, 0)),
            pl.BlockSpec((_R, D_OUT), lambda i: (i, 0)),
        ],
        out_shape=[
            jax.ShapeDtypeStruct((_NPAD, D_OUT), jnp.float32),
            jax.ShapeDtypeStruct((_NPAD, D_OUT), jnp.float32),
        ],
    )(agg2p, agg2p, invdeg, r2, b2, W3l, W3r)


def _mm3_body(a0_ref, a1_ref, invd_ref, r_ref, b_ref, out_ref):
    out_ref[...] = ((a0_ref[0] + a1_ref[0]) * invd_ref[...]
                    + r_ref[...] + b_ref[...])


def _tc_layer3(agg3p, invdeg, r3, b3):
    return pl.pallas_call(
        _mm3_body,
        grid=(_NPAD // _R,),
        in_specs=[
            pl.BlockSpec((1, _R, D_OUT), lambda i: (0, i, 0)),
            pl.BlockSpec((1, _R, D_OUT), lambda i: (1, i, 0)),
            pl.BlockSpec((_R, 1), lambda i: (i, 0)),
            pl.BlockSpec((_R, D_OUT), lambda i: (i, 0)),
            pl.BlockSpec((1, D_OUT), lambda i: (0, 0)),
        ],
        out_specs=pl.BlockSpec((_R, D_OUT), lambda i: (i, 0)),
        out_shape=jax.ShapeDtypeStruct((_NPAD, D_OUT), jnp.float32),
    )(agg3p, agg3p, invdeg, r3, b3)


def kernel(x, edge_index, W1, a1_src, a1_dst, b1, W2l, W2r, b2, W3l, W3r, b3):
    src = edge_index[0]
    dst = edge_index[1]
    pad = _EROWS * _CH - E
    src2d = jnp.pad(src, (0, pad)).reshape(_EROWS, _CH)
    dst2d = jnp.pad(dst, (0, pad)).reshape(_EROWS, _CH)
    zeros2d = jnp.zeros((_NPAD, D_H2), jnp.float32)
    zeros1d = jnp.zeros((_SPAD,), jnp.float32)

    va = W1 @ a1_src
    vb = W1 @ a1_dst
    Vp = jnp.zeros((D_IN, D_IN), jnp.float32)
    Vp = Vp.at[:, 0].set(va).at[:, 1].set(vb)
    alr = _tc_proj(x, Vp)
    al = alr[:, 0]
    ar = alr[:, 1]

    agg1p, s0, s1, d0, d1 = _gat_sc(x, al, ar, src2d, dst2d, zeros2d,
                                    zeros1d)
    s = s0[:_NPAD] + s1[:_NPAD]
    deg = d0[:_NPAD] + d1[:_NPAD]
    inv_s = (1.0 / (s + 1e-16))[:, None]
    invdeg = (1.0 / jnp.maximum(deg, 1.0))[:, None]

    p2, r2 = _tc_layer1(agg1p, inv_s, W1, b1[None, :], W2l, W2r)

    agg2p = _spmm_sc(p2, src2d, dst2d, zeros2d)
    p3, r3 = _tc_layer2(agg2p, invdeg, r2, b2[None, :], W3l, W3r)

    agg3p = _spmm_sc(p3, src2d, dst2d, zeros2d)
    out3 = _tc_layer3(agg3p, invdeg, r3, b3[None, :])
    return out3[:N]
